# Initial kernel scaffold; baseline (speedup 1.0000x reference)
import jax
import jax.numpy as jnp
from jax.experimental import pallas as pl


def kernel(pred_logits, pred_quads, orig_target_sizes):
    raise NotImplementedError("write your pallas kernel here")


# speedup vs baseline: 5.6879x; 5.6879x over previous
import functools

import jax
import jax.numpy as jnp
from jax import lax
from jax.experimental import pallas as pl
from jax.experimental.pallas import tpu as pltpu
from jax.experimental.pallas import tpu_sc as plsc

B = 16
NQ = 20000
NCLS = 36
K = 300
FLAT = NQ * NCLS
HALF = FLAT // 2
W = 12000
NW_HALF = HALF // W
NW_FULL = FLAT // W
VPW = W // 16
UNR = 6
NBUCK = 2048
CAP = 2048
CAPB = CAP + 16
CAPW = 384
OUTK = 512


def _body(logits_ref, quads_ref,
          scores_out, labels_out, quads_out,
          wbuf, strip, hist_l, hist_p, candk, candi, othk, othi, pdata,
          wsc, wlb, wqr, wvl, psc, plb_, pqr, pvl, gidx, qbuf,
          sp_hist, sp_candk, sp_candi, sp_cnt,
          sp_wsc, sp_wlb, sp_wqr, sp_wvl, sem):
  i32 = jnp.int32
  c = lax.axis_index("c")
  s = lax.axis_index("s")
  batch = c * 8 + s // 2
  half = s % 2
  tid = c * 16 + s
  I16 = lax.iota(i32, 16)
  ZERO = jnp.zeros((16,), i32)
  ONES = jnp.ones((16,), i32)
  KS = jnp.full((16,), K, i32)

  def key_of(x):
    bits = lax.bitcast_convert_type(x, i32)
    m = lax.shift_right_arithmetic(bits, jnp.full((16,), 31, i32))
    return lax.bitwise_xor(bits, lax.bitwise_and(m, jnp.full((16,), 0x7FFFFFFF, i32)))

  def sra(v, n):
    return lax.shift_right_arithmetic(v, jnp.full((16,), n, i32))

  def zero_strip():
    def zb(i, _):
      for u in range(8):
        strip[pl.ds((i * 8 + u) * 16, 16)] = ZERO
      return 0
    lax.fori_loop(0, NBUCK * 16 // (16 * 8), zb, 0)

  def hist_stream(lo, n_windows, bucket_fn):
    def wb(w, _):
      pltpu.sync_copy(logits_ref.at[pl.ds(batch * FLAT + lo + w * W, W)], wbuf)
      def vb(vi, _2):
        for u in range(UNR):
          off = (vi * UNR + u) * 16
          k = key_of(wbuf[pl.ds(off, 16)])
          b, msk = bucket_fn(k)
          plsc.addupdate_scatter(strip, [b * 16 + I16], ONES, mask=msk)
        return 0
      lax.fori_loop(0, VPW // UNR, vb, 0)
      return 0
    lax.fori_loop(0, n_windows, wb, 0)

  def strip_reduce(nbuck):
    def g(i, _):
      acc = ZERO
      for lane in range(16):
        acc = acc + plsc.load_gather(strip, [i * 256 + I16 * 16 + lane])
      hist_l[pl.ds(i * 16, 16)] = acc
      return 0
    lax.fori_loop(0, nbuck // 16, g, 0)

  def take(a, j):
    return a.at[j].get(mode="promise_in_bounds")

  def scan_top(nbuck, kneed):
    def sb(i, carry):
      found, bsel, n_ge, n_gt, csum = carry
      v = (nbuck // 16 - 1) - i
      h = hist_l[pl.ds(v * 16, 16)]
      rev = lax.rev(h, (0,))
      cum = plsc.cumsum(rev) + csum
      m = jnp.logical_and(cum >= kneed, found == 0)
      anyc = plsc.all_reduce_population_count(m)
      j = jnp.where(anyc > 0, plsc.all_reduce_ffs(m), 0)
      cum_at = take(cum, j)
      rev_at = take(rev, j)
      hit = anyc > 0
      bsel = jnp.where(hit, v * 16 + 15 - j, bsel)
      n_ge = jnp.where(hit, cum_at, n_ge)
      n_gt = jnp.where(hit, cum_at - rev_at, n_gt)
      found = jnp.where(hit, ONES, found)
      csum = take(cum, jnp.full((16,), 15, i32))
      return found, bsel, n_ge, n_gt, csum
    _, bsel, n_ge, n_gt, _ = lax.fori_loop(
        0, nbuck // 16, sb, (ZERO, ZERO, ZERO, ZERO, ZERO))
    return bsel, n_ge, n_gt

  zero_strip()
  hist_stream(half * HALF, NW_HALF, lambda k: (sra(k, 21) + 1024, None))
  strip_reduce(NBUCK)
  pltpu.sync_copy(hist_l, sp_hist.at[s])
  plsc.subcore_barrier()
  pltpu.sync_copy(sp_hist.at[s ^ 1], hist_p)

  def cmb(i, _):
    for u in range(4):
      off = (i * 4 + u) * 16
      hist_l[pl.ds(off, 16)] = hist_l[pl.ds(off, 16)] + hist_p[pl.ds(off, 16)]
    return 0
  lax.fori_loop(0, NBUCK // (16 * 4), cmb, 0)

  b1, n_ge1, n_gt1 = scan_top(NBUCK, KS)

  t_a = jnp.left_shift(b1 - 1024, jnp.full((16,), 21, i32))
  pdata[pl.ds(0, 16)] = t_a - 1
  pdata[pl.ds(16, 16)] = t_a - 1
  pdata[pl.ds(32, 16)] = ZERO

  mb1 = jnp.max(n_ge1) > CAP

  @pl.when(mb1)
  def _refine1():
    k1 = KS - n_gt1
    pref1 = b1 - 1024
    zero_strip()
    hist_stream(0, NW_FULL,
                lambda k: (lax.bitwise_and(sra(k, 10), jnp.full((16,), 0x7FF, i32)),
                           sra(k, 21) == pref1))
    strip_reduce(NBUCK)
    b2, n_ge2, n_gt2 = scan_top(NBUCK, k1)
    pref21 = pref1 * 2048 + b2
    t2 = pref21 * 1024
    pdata[pl.ds(0, 16)] = t2 - 1
    pdata[pl.ds(16, 16)] = t2 - 1
    pdata[pl.ds(32, 16)] = ZERO
    mb2 = jnp.max(n_gt1 + n_ge2) > CAP

    @pl.when(mb2)
    def _refine2():
      k2 = k1 - n_gt2
      zero_strip()
      hist_stream(0, NW_FULL,
                  lambda k: (lax.bitwise_and(k, jnp.full((16,), 0x3FF, i32)),
                             sra(k, 10) == pref21))
      strip_reduce(1024)
      b3, _, n_gt3 = scan_top(1024, k2)
      t_exact = pref21 * 1024 + b3
      pdata[pl.ds(0, 16)] = t_exact
      pdata[pl.ds(16, 16)] = t_exact
      pdata[pl.ds(32, 16)] = k2 - n_gt3

  t_strict = pdata[pl.ds(0, 16)]
  t_eq = pdata[pl.ds(16, 16)]
  eq_cap = pdata[pl.ds(32, 16)]

  def cwb(w, carry):
    cnt, eqc = carry
    pltpu.sync_copy(logits_ref.at[pl.ds(batch * FLAT + half * HALF + w * W, W)], wbuf)
    def cv(vi, carry2):
      cnt, eqc = carry2
      for u in range(UNR):
        off = (vi * UNR + u) * 16
        k = key_of(wbuf[pl.ds(off, 16)])
        idx = half * HALF + w * W + off + I16
        m1 = k > t_strict
        m2 = jnp.logical_and(k == t_eq, eq_cap > 0)
        pc2 = plsc.cumsum(m2.astype(i32))
        keep2 = jnp.logical_and(m2, (eqc + pc2) <= eq_cap)
        eqc = eqc + plsc.all_reduce_population_count(m2)
        m = jnp.logical_or(m1, keep2)
        mi = m.astype(i32)
        pos = cnt + plsc.cumsum(mi) - mi
        plsc.store_scatter(candk, [pos], k, mask=m)
        plsc.store_scatter(candi, [pos], idx, mask=m)
        cnt = cnt + plsc.all_reduce_population_count(m)
      return cnt, eqc
    return lax.fori_loop(0, VPW // UNR, cv, (cnt, eqc))

  cnt, _ = lax.fori_loop(0, NW_HALF, cwb, (ZERO, ZERO))

  pdata[pl.ds(48, 16)] = cnt
  pltpu.sync_copy(candk, sp_candk.at[s])
  pltpu.sync_copy(candi, sp_candi.at[s])
  pltpu.sync_copy(pdata.at[pl.ds(48, 16)], sp_cnt.at[s])
  plsc.subcore_barrier()
  pltpu.sync_copy(sp_candk.at[s ^ 1], othk)
  pltpu.sync_copy(sp_candi.at[s ^ 1], othi)
  pltpu.sync_copy(sp_cnt.at[s ^ 1], pdata.at[pl.ds(48, 16)])
  cnt_oth = pdata[pl.ds(48, 16)]

  cn_own = jnp.max(cnt)
  cn_oth = jnp.max(cnt_oth)
  nvo = (cn_own + 15) // 16

  def pf(i, _):
    wvl[pl.ds(i * 16, 16)] = ZERO
    wqr[pl.ds(i * 16, 16)] = ZERO
    return 0
  lax.fori_loop(0, CAPW // 16, pf, 0)

  def rank_vo(vo, _):
    base = vo * 16
    kv = candk[pl.ds(base, 16)]
    iv = candi[pl.ds(base, 16)]
    valid = (base + I16) < cnt

    def rj(kref, iref):
      def step(j, r):
        jv = jnp.full((16,), j, i32)
        bk = plsc.load_gather(kref, [jv])
        bi = plsc.load_gather(iref, [jv])
        beats = jnp.logical_or(bk > kv, jnp.logical_and(bk == kv, bi < iv))
        return r + beats.astype(i32)
      return step

    r = lax.fori_loop(0, cn_own, rj(candk, candi), ZERO)
    r = lax.fori_loop(0, cn_oth, rj(othk, othi), r)
    win = jnp.logical_and(valid, r < KS)

    bits = lax.bitwise_xor(
        kv, lax.bitwise_and(sra(kv, 31), jnp.full((16,), 0x7FFFFFFF, i32)))
    x = lax.bitcast_convert_type(bits, jnp.float32)
    sc = 1.0 / (1.0 + jnp.exp(-x))
    qi = iv // NCLS
    lb = iv - qi * NCLS
    plsc.store_scatter(wsc, [r], sc, mask=win)
    plsc.store_scatter(wlb, [r], lb, mask=win)
    plsc.store_scatter(wqr, [r], batch * NQ + qi, mask=win)
    plsc.store_scatter(wvl, [r], ONES, mask=win)
    return 0
  lax.fori_loop(0, nvo, rank_vo, 0)

  pltpu.sync_copy(wsc, sp_wsc.at[s])
  pltpu.sync_copy(wlb, sp_wlb.at[s])
  pltpu.sync_copy(wqr, sp_wqr.at[s])
  pltpu.sync_copy(wvl, sp_wvl.at[s])
  plsc.subcore_barrier()

  @pl.when(half == 0)
  def _writeout():
    pltpu.sync_copy(sp_wsc.at[s + 1], psc)
    pltpu.sync_copy(sp_wlb.at[s + 1], plb_)
    pltpu.sync_copy(sp_wqr.at[s + 1], pqr)
    pltpu.sync_copy(sp_wvl.at[s + 1], pvl)

    def mg(i, _):
      sl = pl.ds(i * 16, 16)
      use_p = pvl[sl] > 0
      wsc[sl] = jnp.where(use_p, psc[sl], wsc[sl])
      wlb[sl] = jnp.where(use_p, plb_[sl], wlb[sl])
      wqr[sl] = jnp.where(use_p, pqr[sl], wqr[sl])
      return 0
    lax.fori_loop(0, CAPW // 16, mg, 0)

    pltpu.sync_copy(wsc, scores_out.at[pl.ds(batch * OUTK, CAPW)])
    pltpu.sync_copy(wlb, labels_out.at[pl.ds(batch * OUTK, CAPW)])
    for chunk in range(CAPW // 128):
      cb = chunk * 128
      for t in range(8):
        gidx[pl.ds(t * 16, 16)] = wqr[pl.ds(cb + t * 16, 16)]
      pltpu.async_copy(quads_ref.at[gidx], qbuf.at[pl.ds(cb, 128)], sem).wait()
    pltpu.sync_copy(qbuf, quads_out.at[pl.ds(batch * OUTK, CAPW)])


@jax.jit
def _sc_topk(logits1d, quads2d):
  f32 = jnp.float32
  i32 = jnp.int32
  mesh = plsc.VectorSubcoreMesh(
      core_axis_name="c", subcore_axis_name="s", num_cores=2, num_subcores=16)
  return pl.kernel(
      _body,
      out_type=(
          jax.ShapeDtypeStruct((B * OUTK,), f32),
          jax.ShapeDtypeStruct((B * OUTK,), i32),
          jax.ShapeDtypeStruct((B * OUTK, 8), f32),
      ),
      mesh=mesh,
      compiler_params=pltpu.CompilerParams(needs_layout_passes=False, use_tc_tiling_on_sc=False),
      scratch_types=[
          pltpu.VMEM((W,), f32),
          pltpu.VMEM((NBUCK * 16,), i32),
          pltpu.VMEM((NBUCK,), i32),
          pltpu.VMEM((NBUCK,), i32),
          pltpu.VMEM((CAPB,), i32),
          pltpu.VMEM((CAPB,), i32),
          pltpu.VMEM((CAPB,), i32),
          pltpu.VMEM((CAPB,), i32),
          pltpu.VMEM((64,), i32),
          pltpu.VMEM((CAPW,), f32),
          pltpu.VMEM((CAPW,), i32),
          pltpu.VMEM((CAPW,), i32),
          pltpu.VMEM((CAPW,), i32),
          pltpu.VMEM((CAPW,), f32),
          pltpu.VMEM((CAPW,), i32),
          pltpu.VMEM((CAPW,), i32),
          pltpu.VMEM((CAPW,), i32),
          pltpu.VMEM((128,), i32),
          pltpu.VMEM((CAPW, 8), f32),
          pltpu.VMEM_SHARED((16, NBUCK), i32),
          pltpu.VMEM_SHARED((16, CAPB), i32),
          pltpu.VMEM_SHARED((16, CAPB), i32),
          pltpu.VMEM_SHARED((16, 16), i32),
          pltpu.VMEM_SHARED((16, CAPW), f32),
          pltpu.VMEM_SHARED((16, CAPW), i32),
          pltpu.VMEM_SHARED((16, CAPW), i32),
          pltpu.VMEM_SHARED((16, CAPW), i32),
          pltpu.SemaphoreType.DMA,
      ],
  )(logits1d, quads2d)


def _scale_body(q_ref, s_ref, o_ref):
  o_ref[...] = q_ref[...] * s_ref[...][:, None, :]


@jax.jit
def _scale_quads(quads_raw, scale8):
  return pl.pallas_call(
      _scale_body,
      out_shape=jax.ShapeDtypeStruct((B, OUTK, 8), jnp.float32),
  )(quads_raw, scale8)


def kernel(pred_logits, pred_quads, orig_target_sizes):
  logits1d = pred_logits.reshape(-1)
  quads2d = pred_quads.reshape(B * NQ, 8)
  scores_p, labels_p, quads_p = _sc_topk(logits1d, quads2d)
  scale8 = jnp.tile(orig_target_sizes, (1, 4))
  quads_s = _scale_quads(quads_p.reshape(B, OUTK, 8), scale8)
  labels = labels_p.reshape(B, OUTK)[:, :K]
  quads = quads_s[:, :K, :]
  scores = scores_p.reshape(B, OUTK)[:, :K]
  return labels, quads, scores
